# Initial kernel scaffold; baseline (speedup 1.0000x reference)
import jax
import jax.numpy as jnp
from jax.experimental import pallas as pl


def kernel(hidden_states, ret_text_embs, ret_input_ids, logits, Wq_attn, bq_attn, Wk_attn, bk_attn, Wv_attn, bv_attn, Wq_ptr, bq_ptr, Wc_ptr, bc_ptr):
    raise NotImplementedError("write your pallas kernel here")


# speedup vs baseline: 5.4192x; 5.4192x over previous
import functools

import jax
import jax.numpy as jnp
from jax import lax
from jax.experimental import pallas as pl
from jax.experimental.pallas import tpu as pltpu
from jax.experimental.pallas import tpu_sc as plsc

_NEG = -1e9


def _tc_body(hs_ref, embs_ref, ids_ref, logits_ref,
             Wq_ref, bq_ref, Wk_ref, bk_ref, Wv_ref, bv_ref,
             Wqp_ref, c0_ref, Wcp_ref,
             base_ref, vals_ref, idx_ref, A_ref):
    b = pl.program_id(0)
    T, d = hs_ref.shape[1], hs_ref.shape[2]
    n_ret, S = embs_ref.shape[1], embs_ref.shape[2]
    V = logits_ref.shape[2]
    f32 = jnp.float32
    dot = lambda x, y: lax.dot_general(
        x, y, (((1,), (0,)), ((), ())), preferred_element_type=f32)
    cT = lambda x, y: lax.dot_general(
        x, y, (((1,), (1,)), ((), ())), preferred_element_type=f32)

    @pl.when(b == 0)
    def _():
        A_ref[...] = cT(Wq_ref[...], Wk_ref[...])

    hs = hs_ref[0]
    bq = bq_ref[...]
    bk = bk_ref[...]
    bv = bv_ref[...]
    Wqp = Wqp_ref[...]
    Wcp = Wcp_ref[...]

    qbk = cT(bk, Wq_ref[...])
    kbq = cT(bq, Wk_ref[...])
    wvc = cT(Wcp, Wv_ref[...])
    t4 = jnp.sum(bq * bk)
    gate_c = c0_ref[0, 0] + jnp.sum(bv * Wcp)

    U = dot(hs, A_ref[...])
    ct = cT(hs, qbk)
    hq = cT(hs, Wqp)
    scale = f32(1.0) / jnp.sqrt(f32(d))
    inv_r = f32(1.0 / n_ret)

    ids_row = ids_ref[0]
    psum = jnp.zeros((T, 1), f32)
    for r in range(n_ret):
        embs_r = embs_ref[0, r]
        ids_r = ids_row[:, r * S:(r + 1) * S]
        scores = (cT(U, embs_r) + ct + cT(kbq, embs_r) + t4) * scale
        scores = jnp.where(ids_r == 0, _NEG, scores)
        m = jnp.max(scores, axis=1, keepdims=True)
        e = jnp.exp(scores - m)
        attn = e / jnp.sum(e, axis=1, keepdims=True)
        vc = cT(wvc, embs_r)
        p = jax.nn.sigmoid(hq + cT(attn, vc) + gate_c)
        psum = psum + p
        vals_ref[0, :, r * S:(r + 1) * S] = (p * inv_r) * attn

    g = f32(1.0) - psum * inv_r

    lg = logits_ref[0]
    lm = jnp.max(lg, axis=1, keepdims=True)
    le = jnp.exp(lg - lm)
    base_ref[0] = (g / jnp.sum(le, axis=1, keepdims=True)) * le

    bb = b % 2
    tt = lax.broadcasted_iota(jnp.int32, (T, n_ret * S), 0)
    idx_ref[0] = bb * (T * V) + tt * V + ids_row


def _tc_call(hs, embs, ids32, logits, Wq, bq2, Wk, bk2, Wv, bv2, Wqp2, c0, Wcp2):
    bsz, T, d = hs.shape
    n_ret, S = embs.shape[1], embs.shape[2]
    V = logits.shape[2]
    f32 = jnp.float32
    full = lambda shp: pl.BlockSpec(shp, lambda b: (0,) * len(shp))
    return pl.pallas_call(
        _tc_body,
        grid=(bsz,),
        in_specs=[
            pl.BlockSpec((1, T, d), lambda b: (b, 0, 0)),
            pl.BlockSpec((1, n_ret, S, d), lambda b: (b, 0, 0, 0)),
            pl.BlockSpec((1, 1, n_ret * S), lambda b: (b, 0, 0)),
            pl.BlockSpec((1, T, V), lambda b: (b, 0, 0)),
            full((d, d)), full((1, d)),
            full((d, d)), full((1, d)),
            full((d, d)), full((1, d)),
            full((1, d)), full((1, 1)), full((1, d)),
        ],
        out_specs=[
            pl.BlockSpec((1, T, V), lambda b: (b, 0, 0)),
            pl.BlockSpec((1, T, n_ret * S), lambda b: (b, 0, 0)),
            pl.BlockSpec((1, T, n_ret * S), lambda b: (b, 0, 0)),
        ],
        out_shape=[
            jax.ShapeDtypeStruct((bsz, T, V), f32),
            jax.ShapeDtypeStruct((bsz, T, n_ret * S), f32),
            jax.ShapeDtypeStruct((bsz, T, n_ret * S), jnp.int32),
        ],
        scratch_shapes=[pltpu.VMEM((d, d), f32)],
    )(hs, embs, ids32, logits, Wq, bq2, Wk, bk2, Wv, bv2, Wqp2, c0, Wcp2)


def _sc_scatter(base_flat, idx2, vals2, bsz, T, V):
    TV = T * V
    QV = (T // 8) * V
    GR, L = idx2.shape
    G = GR // bsz
    GPQ = G // 8
    bpc = bsz // 2
    mesh = plsc.VectorSubcoreMesh(core_axis_name="c", subcore_axis_name="s")

    @functools.partial(
        pl.kernel,
        out_type=jax.ShapeDtypeStruct((bsz * TV,), jnp.float32),
        mesh=mesh,
        scratch_types=[
            pltpu.VMEM_SHARED((2 * TV,), jnp.float32),
            pltpu.VMEM((GPQ, L), jnp.int32),
            pltpu.VMEM((GPQ, L), jnp.float32),
        ],
    )
    def sc_fn(base_hbm, idx_hbm, vals_hbm, out_hbm, spmem, idx_v, vals_v):
        c = lax.axis_index("c")
        s = lax.axis_index("s")
        bbp = s // 8
        q = s % 8
        for p in range(2):
            b = c * bpc + p * 2 + bbp
            hb = b * TV + q * QV
            sp = bbp * TV + q * QV
            row0 = b * G + q * GPQ
            pltpu.sync_copy(base_hbm.at[pl.ds(hb, QV)], spmem.at[pl.ds(sp, QV)])
            pltpu.sync_copy(idx_hbm.at[pl.ds(row0, GPQ)], idx_v)
            pltpu.sync_copy(vals_hbm.at[pl.ds(row0, GPQ)], vals_v)
            for g in range(GPQ):
                pltpu.sync_copy(vals_v.at[g], spmem.at[idx_v.at[g]], add=True)
            pltpu.sync_copy(spmem.at[pl.ds(sp, QV)], out_hbm.at[pl.ds(hb, QV)])

    return sc_fn(base_flat, idx2, vals2)


def kernel(hidden_states, ret_text_embs, ret_input_ids, logits,
           Wq_attn, bq_attn, Wk_attn, bk_attn, Wv_attn, bv_attn,
           Wq_ptr, bq_ptr, Wc_ptr, bc_ptr):
    bsz, n_ret, S = ret_input_ids.shape
    T = hidden_states.shape[1]
    d = hidden_states.shape[2]
    V = logits.shape[-1]
    f32 = jnp.float32

    ids32 = ret_input_ids.astype(jnp.int32).reshape(bsz, 1, n_ret * S)
    bq2 = bq_attn.astype(f32).reshape(1, d)
    bk2 = bk_attn.astype(f32).reshape(1, d)
    bv2 = bv_attn.astype(f32).reshape(1, d)
    Wqp2 = Wq_ptr.astype(f32).reshape(1, d)
    Wcp2 = Wc_ptr.astype(f32).reshape(1, d)
    c0 = (bq_ptr + bc_ptr).astype(f32).reshape(1, 1)

    base, vals, sidx = _tc_call(
        hidden_states.astype(f32), ret_text_embs.astype(f32), ids32,
        logits.astype(f32), Wq_attn.astype(f32), bq2, Wk_attn.astype(f32),
        bk2, Wv_attn.astype(f32), bv2, Wqp2, c0, Wcp2)

    L = 100
    GR = (bsz * T * n_ret * S) // L
    out = _sc_scatter(base.reshape(bsz * T * V),
                      sidx.reshape(GR, L), vals.reshape(GR, L), bsz, T, V)
    return out.reshape(bsz, T, V)


# speedup vs baseline: 6.9800x; 1.2880x over previous
import functools

import jax
import jax.numpy as jnp
from jax import lax
from jax.experimental import pallas as pl
from jax.experimental.pallas import tpu as pltpu
from jax.experimental.pallas import tpu_sc as plsc

_NEG = -1e9


def _tc_body(hs_ref, embs_ref, ids_ref, logits_ref,
             Wq_ref, bq_ref, Wk_ref, bk_ref, Wv_ref, bv_ref,
             Wqp_ref, c0_ref, Wcp_ref,
             base_ref, vals_ref, idx_ref, A_ref):
    b = pl.program_id(0)
    T, d = hs_ref.shape[1], hs_ref.shape[2]
    n_ret, S = embs_ref.shape[1], embs_ref.shape[2]
    f32 = jnp.float32
    dot = lambda x, y: lax.dot_general(
        x, y, (((1,), (0,)), ((), ())), preferred_element_type=f32)
    cT = lambda x, y: lax.dot_general(
        x, y, (((1,), (1,)), ((), ())), preferred_element_type=f32)

    @pl.when(b == 0)
    def _():
        A_ref[...] = cT(Wq_ref[...], Wk_ref[...])

    hs = hs_ref[0]
    bq = bq_ref[...]
    bk = bk_ref[...]
    bv = bv_ref[...]
    Wqp = Wqp_ref[...]
    Wcp = Wcp_ref[...]

    qbk = cT(bk, Wq_ref[...])
    kbq = cT(bq, Wk_ref[...])
    wvc = cT(Wcp, Wv_ref[...])
    t4 = jnp.sum(bq * bk)
    gate_c = c0_ref[0, 0] + jnp.sum(bv * Wcp)

    U = dot(hs, A_ref[...])
    ct = cT(hs, qbk)
    hq = cT(hs, Wqp)
    scale = f32(1.0) / jnp.sqrt(f32(d))
    inv_r = f32(1.0 / n_ret)

    ids_row = ids_ref[0]
    psum = jnp.zeros((T, 1), f32)
    for r in range(n_ret):
        embs_r = embs_ref[0, r]
        ids_r = ids_row[:, r * S:(r + 1) * S]
        scores = (cT(U, embs_r) + ct + cT(kbq, embs_r) + t4) * scale
        scores = jnp.where(ids_r == 0, _NEG, scores)
        m = jnp.max(scores, axis=1, keepdims=True)
        e = jnp.exp(scores - m)
        attn = e / jnp.sum(e, axis=1, keepdims=True)
        vc = cT(wvc, embs_r)
        p = jax.nn.sigmoid(hq + cT(attn, vc) + gate_c)
        psum = psum + p
        v_r = (p * inv_r) * attn
        vals_ref[0, pl.ds((2 * r) * T, T)] = v_r[:, :S // 2]
        vals_ref[0, pl.ds((2 * r + 1) * T, T)] = v_r[:, S // 2:]
        V = base_ref.shape[2]
        tt = lax.broadcasted_iota(jnp.int32, (T, S), 0)
        full_idx = (b % 2) * (T * V) + tt * V + ids_r
        idx_ref[0, pl.ds((2 * r) * T, T)] = full_idx[:, :S // 2]
        idx_ref[0, pl.ds((2 * r + 1) * T, T)] = full_idx[:, S // 2:]

    g = f32(1.0) - psum * inv_r

    lg = logits_ref[0]
    lm = jnp.max(lg, axis=1, keepdims=True)
    le = jnp.exp(lg - lm)
    base_ref[0] = (g / jnp.sum(le, axis=1, keepdims=True)) * le


def _tc_call(hs, embs, ids32, logits, Wq, bq2, Wk, bk2, Wv, bv2, Wqp2, c0, Wcp2):
    bsz, T, d = hs.shape
    n_ret, S = embs.shape[1], embs.shape[2]
    V = logits.shape[2]
    f32 = jnp.float32
    nk = 2 * n_ret
    full = lambda shp: pl.BlockSpec(shp, lambda b: (0,) * len(shp))
    return pl.pallas_call(
        _tc_body,
        grid=(bsz,),
        in_specs=[
            pl.BlockSpec((1, T, d), lambda b: (b, 0, 0)),
            pl.BlockSpec((1, n_ret, S, d), lambda b: (b, 0, 0, 0)),
            pl.BlockSpec((1, 1, n_ret * S), lambda b: (b, 0, 0)),
            pl.BlockSpec((1, T, V), lambda b: (b, 0, 0)),
            full((d, d)), full((1, d)),
            full((d, d)), full((1, d)),
            full((d, d)), full((1, d)),
            full((1, d)), full((1, 1)), full((1, d)),
        ],
        out_specs=[
            pl.BlockSpec((1, T, V), lambda b: (b, 0, 0)),
            pl.BlockSpec((1, nk * T, S // 2), lambda b: (b, 0, 0)),
            pl.BlockSpec((1, nk * T, S // 2), lambda b: (b, 0, 0)),
        ],
        out_shape=[
            jax.ShapeDtypeStruct((bsz, T, V), f32),
            jax.ShapeDtypeStruct((bsz, nk * T, S // 2), f32),
            jax.ShapeDtypeStruct((bsz, nk * T, S // 2), jnp.int32),
        ],
        scratch_shapes=[pltpu.VMEM((d, d), f32)],
    )(hs, embs, ids32, logits, Wq, bq2, Wk, bk2, Wv, bv2, Wqp2, c0, Wcp2)


def _sc_scatter(base3, idx4, vals4, bsz, T, V):
    NG, L = vals4.shape
    NGB = NG // bsz
    bpc = bsz // 2
    mesh = plsc.VectorSubcoreMesh(core_axis_name="c", subcore_axis_name="s")

    @functools.partial(
        pl.kernel,
        out_type=jax.ShapeDtypeStruct((bsz, T, V), jnp.float32),
        mesh=mesh,
        scratch_types=[
            pltpu.VMEM_SHARED((2 * T * V,), jnp.float32),
            pltpu.VMEM((T, L), jnp.int32),
            pltpu.VMEM((T, L), jnp.float32),
        ],
        name="sc_scatter",
    )
    def sc_fn(base_hbm, idx_hbm, vals_hbm, out_hbm, spmem, idx_v, vals_v):
        c = lax.axis_index("c")
        s = lax.axis_index("s")
        bbp = s // 8
        q = s % 8
        kp = s % 8
        for p in range(2):
            b = c * bpc + p * 2 + bbp
            for i in range(2):
                t = 2 * q + i
                pltpu.sync_copy(
                    base_hbm.at[b, t],
                    spmem.at[pl.ds((bbp * T + t) * V, V)])
            grow = b * NGB + kp * T
            pltpu.sync_copy(idx_hbm.at[pl.ds(grow, T)], idx_v)
            pltpu.sync_copy(vals_hbm.at[pl.ds(grow, T)], vals_v)
            plsc.subcore_barrier()
            for g in range(T):
                pltpu.sync_copy(vals_v.at[g], spmem.at[idx_v.at[g]],
                                add=True)
            plsc.subcore_barrier()
            for i in range(2):
                t = 2 * q + i
                pltpu.sync_copy(
                    spmem.at[pl.ds((bbp * T + t) * V, V)],
                    out_hbm.at[b, t])

    return sc_fn(base3, idx4, vals4)


def kernel(hidden_states, ret_text_embs, ret_input_ids, logits,
           Wq_attn, bq_attn, Wk_attn, bk_attn, Wv_attn, bv_attn,
           Wq_ptr, bq_ptr, Wc_ptr, bc_ptr):
    bsz, n_ret, S = ret_input_ids.shape
    T = hidden_states.shape[1]
    d = hidden_states.shape[2]
    V = logits.shape[-1]
    f32 = jnp.float32

    ids32 = ret_input_ids.astype(jnp.int32).reshape(bsz, 1, n_ret * S)
    bq2 = bq_attn.astype(f32).reshape(1, d)
    bk2 = bk_attn.astype(f32).reshape(1, d)
    bv2 = bv_attn.astype(f32).reshape(1, d)
    Wqp2 = Wq_ptr.astype(f32).reshape(1, d)
    Wcp2 = Wc_ptr.astype(f32).reshape(1, d)
    c0 = (bq_ptr + bc_ptr).astype(f32).reshape(1, 1)

    base, vals, sidx = _tc_call(
        hidden_states.astype(f32), ret_text_embs.astype(f32), ids32,
        logits.astype(f32), Wq_attn.astype(f32), bq2, Wk_attn.astype(f32),
        bk2, Wv_attn.astype(f32), bv2, Wqp2, c0, Wcp2)

    NG = vals.shape[1]
    L = vals.shape[2]
    return _sc_scatter(base, sidx.reshape(bsz * NG, L),
                       vals.reshape(bsz * NG, L), bsz, T, V)


# speedup vs baseline: 7.5045x; 1.0751x over previous
import functools

import jax
import jax.numpy as jnp
from jax import lax
from jax.experimental import pallas as pl
from jax.experimental.pallas import tpu as pltpu
from jax.experimental.pallas import tpu_sc as plsc

_NEG = -1e9


def _tc_body(hs_ref, embs_ref, ids_ref, logits_ref,
             Wq_ref, bq_ref, Wk_ref, bk_ref, Wv_ref, bv_ref,
             Wqp_ref, c0_ref, Wcp_ref,
             base_ref, vals_ref, idx_ref, A_ref):
    b = pl.program_id(0)
    T, d = hs_ref.shape[1], hs_ref.shape[2]
    n_ret, S = embs_ref.shape[1], embs_ref.shape[2]
    f32 = jnp.float32
    dot = lambda x, y: lax.dot_general(
        x, y, (((1,), (0,)), ((), ())), preferred_element_type=f32)
    cT = lambda x, y: lax.dot_general(
        x, y, (((1,), (1,)), ((), ())), preferred_element_type=f32)

    @pl.when(b == 0)
    def _():
        A_ref[...] = cT(Wq_ref[...], Wk_ref[...])

    hs = hs_ref[0]
    bq = bq_ref[...]
    bk = bk_ref[...]
    bv = bv_ref[...]
    Wqp = Wqp_ref[...]
    Wcp = Wcp_ref[...]

    qbk = cT(bk, Wq_ref[...])
    kbq = cT(bq, Wk_ref[...])
    wvc = cT(Wcp, Wv_ref[...])
    t4 = jnp.sum(bq * bk)
    gate_c = c0_ref[0, 0] + jnp.sum(bv * Wcp)

    U = dot(hs, A_ref[...])
    ct = cT(hs, qbk)
    hq = cT(hs, Wqp)
    scale = f32(1.0) / jnp.sqrt(f32(d))
    inv_r = f32(1.0 / n_ret)

    ids_row = ids_ref[0]
    psum = jnp.zeros((T, 1), f32)
    for r in range(n_ret):
        embs_r = embs_ref[0, r]
        ids_r = ids_row[:, r * S:(r + 1) * S]
        scores = (cT(U, embs_r) + ct + cT(kbq, embs_r) + t4) * scale
        scores = jnp.where(ids_r == 0, _NEG, scores)
        m = jnp.max(scores, axis=1, keepdims=True)
        e = jnp.exp(scores - m)
        attn = e / jnp.sum(e, axis=1, keepdims=True)
        vc = cT(wvc, embs_r)
        p = jax.nn.sigmoid(hq + cT(attn, vc) + gate_c)
        psum = psum + p
        v_r = (p * inv_r) * attn
        vals_ref[0, pl.ds((2 * r) * T, T)] = v_r[:, :S // 2]
        vals_ref[0, pl.ds((2 * r + 1) * T, T)] = v_r[:, S // 2:]
        V = base_ref.shape[2]
        tt = lax.broadcasted_iota(jnp.int32, (T, S), 0)
        full_idx = (b % 2) * (T * V) + tt * V + ids_r
        idx_ref[0, pl.ds((2 * r) * T, T)] = full_idx[:, :S // 2]
        idx_ref[0, pl.ds((2 * r + 1) * T, T)] = full_idx[:, S // 2:]

    g = f32(1.0) - psum * inv_r

    lg = logits_ref[0]
    lm = jnp.max(lg, axis=1, keepdims=True)
    le = jnp.exp(lg - lm)
    base_ref[0] = (g / jnp.sum(le, axis=1, keepdims=True)) * le


def _tc_call(hs, embs, ids32, logits, Wq, bq2, Wk, bk2, Wv, bv2, Wqp2, c0, Wcp2):
    bsz, T, d = hs.shape
    n_ret, S = embs.shape[1], embs.shape[2]
    V = logits.shape[2]
    f32 = jnp.float32
    nk = 2 * n_ret
    full = lambda shp: pl.BlockSpec(shp, lambda b: (0,) * len(shp))
    return pl.pallas_call(
        _tc_body,
        grid=(bsz,),
        in_specs=[
            pl.BlockSpec((1, T, d), lambda b: (b, 0, 0)),
            pl.BlockSpec((1, n_ret, S, d), lambda b: (b, 0, 0, 0)),
            pl.BlockSpec((1, 1, n_ret * S), lambda b: (b, 0, 0)),
            pl.BlockSpec((1, T, V), lambda b: (b, 0, 0)),
            full((d, d)), full((1, d)),
            full((d, d)), full((1, d)),
            full((d, d)), full((1, d)),
            full((1, d)), full((1, 1)), full((1, d)),
        ],
        out_specs=[
            pl.BlockSpec((1, T, V), lambda b: (b, 0, 0)),
            pl.BlockSpec((1, nk * T, S // 2), lambda b: (b, 0, 0)),
            pl.BlockSpec((1, nk * T, S // 2), lambda b: (b, 0, 0)),
        ],
        out_shape=[
            jax.ShapeDtypeStruct((bsz, T, V), f32),
            jax.ShapeDtypeStruct((bsz, nk * T, S // 2), f32),
            jax.ShapeDtypeStruct((bsz, nk * T, S // 2), jnp.int32),
        ],
        scratch_shapes=[pltpu.VMEM((d, d), f32)],
    )(hs, embs, ids32, logits, Wq, bq2, Wk, bk2, Wv, bv2, Wqp2, c0, Wcp2)


def _sc_scatter(base3, idx4, vals4, bsz, T, V):
    NG, L = vals4.shape
    NGB = NG // bsz
    bpc = bsz // 2
    mesh = plsc.VectorSubcoreMesh(core_axis_name="c", subcore_axis_name="s")

    @functools.partial(
        pl.kernel,
        out_type=jax.ShapeDtypeStruct((bsz, T, V), jnp.float32),
        mesh=mesh,
        scratch_types=[
            pltpu.VMEM_SHARED((2 * T * V,), jnp.float32),
            pltpu.VMEM((T, L), jnp.int32),
            pltpu.VMEM((T, L), jnp.float32),
            pltpu.SemaphoreType.DMA,
        ],
        name="sc_scatter",
    )
    def sc_fn(base_hbm, idx_hbm, vals_hbm, out_hbm, spmem, idx_v, vals_v,
              sem):
        c = lax.axis_index("c")
        s = lax.axis_index("s")
        bbp = s // 8
        q = s % 8
        kp = s % 8
        for p in range(2):
            b = c * bpc + p * 2 + bbp
            grow = b * NGB + kp * T
            ds = [pltpu.async_copy(
                      base_hbm.at[b, 2 * q + i],
                      spmem.at[pl.ds((bbp * T + 2 * q + i) * V, V)], sem)
                  for i in range(2)]
            ds.append(pltpu.async_copy(idx_hbm.at[pl.ds(grow, T)], idx_v,
                                       sem))
            ds.append(pltpu.async_copy(vals_hbm.at[pl.ds(grow, T)], vals_v,
                                       sem))
            for dd in ds:
                dd.wait()
            plsc.subcore_barrier()
            ds = [pltpu.async_copy(vals_v.at[g], spmem.at[idx_v.at[g]],
                                   sem, add=True)
                  for g in range(T)]
            for dd in ds:
                dd.wait()
            plsc.subcore_barrier()
            ds = [pltpu.async_copy(
                      spmem.at[pl.ds((bbp * T + 2 * q + i) * V, V)],
                      out_hbm.at[b, 2 * q + i], sem)
                  for i in range(2)]
            for dd in ds:
                dd.wait()

    return sc_fn(base3, idx4, vals4)


def kernel(hidden_states, ret_text_embs, ret_input_ids, logits,
           Wq_attn, bq_attn, Wk_attn, bk_attn, Wv_attn, bv_attn,
           Wq_ptr, bq_ptr, Wc_ptr, bc_ptr):
    bsz, n_ret, S = ret_input_ids.shape
    T = hidden_states.shape[1]
    d = hidden_states.shape[2]
    V = logits.shape[-1]
    f32 = jnp.float32

    ids32 = ret_input_ids.astype(jnp.int32).reshape(bsz, 1, n_ret * S)
    bq2 = bq_attn.astype(f32).reshape(1, d)
    bk2 = bk_attn.astype(f32).reshape(1, d)
    bv2 = bv_attn.astype(f32).reshape(1, d)
    Wqp2 = Wq_ptr.astype(f32).reshape(1, d)
    Wcp2 = Wc_ptr.astype(f32).reshape(1, d)
    c0 = (bq_ptr + bc_ptr).astype(f32).reshape(1, 1)

    base, vals, sidx = _tc_call(
        hidden_states.astype(f32), ret_text_embs.astype(f32), ids32,
        logits.astype(f32), Wq_attn.astype(f32), bq2, Wk_attn.astype(f32),
        bk2, Wv_attn.astype(f32), bv2, Wqp2, c0, Wcp2)

    NG = vals.shape[1]
    L = vals.shape[2]
    return _sc_scatter(base, sidx.reshape(bsz * NG, L),
                       vals.reshape(bsz * NG, L), bsz, T, V)
